# baseline (speedup 1.0000x reference)
import functools

import jax
import jax.numpy as jnp
from jax import lax
from jax.experimental import pallas as pl
from jax.experimental.pallas import tpu as pltpu
from jax.experimental.pallas import tpu_sc as plsc

_ROWS_PER_BLOCK = 4096

_SC_INFO = plsc.get_sparse_core_info()
_NC = _SC_INFO.num_cores
_NS = _SC_INFO.num_subcores
_NW = _NC * _NS
_IDX_CHUNK = 128


def _vq_block(x_ref, e_ref, ind_ref, acc_ref):
    i = pl.program_id(0)
    x = x_ref[...]
    e = e_ref[...]
    s = jax.lax.dot_general(
        x * -2.0, e, (((1,), (0,)), ((), ())),
        preferred_element_type=jnp.float32,
    )
    x2 = jnp.sum(x * x, axis=1, keepdims=True)
    e2 = jnp.sum(e * e, axis=0, keepdims=True)
    dist = x2 + s + e2
    ind = jnp.argmin(dist, axis=1).astype(jnp.int32)
    mind = jnp.min(dist, axis=1)
    ind_ref[0, 0, :] = ind

    @pl.when(i == 0)
    def _init():
        acc_ref[...] = jnp.zeros((1, 1), jnp.float32)

    acc_ref[...] += jnp.sum(mind).reshape(1, 1)


def _tc_argmin(x, embed):
    n, dim = x.shape
    k = embed.shape[1]
    nblocks = n // _ROWS_PER_BLOCK
    ind3, acc = pl.pallas_call(
        _vq_block,
        grid=(nblocks,),
        in_specs=[
            pl.BlockSpec((_ROWS_PER_BLOCK, dim), lambda i: (i, 0)),
            pl.BlockSpec((dim, k), lambda i: (0, 0)),
        ],
        out_specs=[
            pl.BlockSpec((1, 1, _ROWS_PER_BLOCK), lambda i: (i, 0, 0)),
            pl.BlockSpec((1, 1), lambda i: (0, 0)),
        ],
        out_shape=[
            jax.ShapeDtypeStruct((nblocks, 1, _ROWS_PER_BLOCK), jnp.int32),
            jax.ShapeDtypeStruct((1, 1), jnp.float32),
        ],
    )(x, embed)
    return ind3.reshape(n), acc


def _sc_gather(table, idx, n, dim):
    bpw = n // _NW
    mesh = plsc.VectorSubcoreMesh(core_axis_name="c", subcore_axis_name="s")

    @functools.partial(
        pl.kernel,
        mesh=mesh,
        out_type=jax.ShapeDtypeStruct((n, dim), jnp.float32),
        scratch_types=[
            pltpu.VMEM((bpw,), jnp.int32),
            pltpu.VMEM((bpw, dim), jnp.float32),
            pltpu.SemaphoreType.DMA,
        ],
    )
    def gather_k(table_hbm, idx_hbm, out_hbm, idx_v, rows_v, sem):
        wid = lax.axis_index("s") * _NC + lax.axis_index("c")
        base = wid * bpw
        pltpu.sync_copy(idx_hbm.at[pl.ds(base, bpw)], idx_v)
        copies = []
        for j in range(bpw // _IDX_CHUNK):
            copies.append(pltpu.async_copy(
                table_hbm.at[idx_v.at[pl.ds(j * _IDX_CHUNK, _IDX_CHUNK)]],
                rows_v.at[pl.ds(j * _IDX_CHUNK, _IDX_CHUNK)],
                sem,
            ))
        for c in copies:
            c.wait()
        pltpu.sync_copy(rows_v, out_hbm.at[pl.ds(base, bpw)])

    return gather_k(table, idx)


@functools.partial(jax.jit, static_argnames=())
def kernel(input, embed):
    n = input.shape[0] * input.shape[1]
    dim = embed.shape[0]
    x = input.reshape(n, dim)
    ind_flat, acc = _tc_argmin(x, embed)
    table = jnp.pad(embed.T, ((0, 0), (0, 128 - dim)))
    q = _sc_gather(table, ind_flat, n, 128)[:, :dim]
    quantize = q.reshape(input.shape)
    embed_ind = ind_flat.reshape(input.shape[:-1])
    diff = (acc[0, 0] / (n * dim)).astype(jnp.float32)
    return (quantize, diff, embed_ind)


# speedup vs baseline: 1.1375x; 1.1375x over previous
import functools

import jax
import jax.numpy as jnp
from jax import lax
from jax.experimental import pallas as pl
from jax.experimental.pallas import tpu as pltpu
from jax.experimental.pallas import tpu_sc as plsc

_ROWS_PER_BLOCK = 4096

_SC_INFO = plsc.get_sparse_core_info()
_NC = _SC_INFO.num_cores
_NS = _SC_INFO.num_subcores
_NW = _NC * _NS
_IDX_CHUNK = 128


def _vq_block(x_ref, e_ref, ind_ref, acc_ref):
    i = pl.program_id(0)
    x = x_ref[...]
    e = e_ref[...]
    s = jax.lax.dot_general(
        x * -2.0, e, (((1,), (0,)), ((), ())),
        preferred_element_type=jnp.float32,
    )
    x2 = jnp.sum(x * x, axis=1, keepdims=True)
    e2 = jnp.sum(e * e, axis=0, keepdims=True)
    dist = x2 + s + e2
    ind = jnp.argmin(dist, axis=1).astype(jnp.int32)
    mind = jnp.min(dist, axis=1)
    ind_ref[0, 0, :] = ind

    @pl.when(i == 0)
    def _init():
        acc_ref[...] = jnp.zeros((1, 1), jnp.float32)

    acc_ref[...] += jnp.sum(mind).reshape(1, 1)


def _tc_argmin(x, embed):
    n, dim = x.shape
    k = embed.shape[1]
    nblocks = n // _ROWS_PER_BLOCK
    ind3, acc = pl.pallas_call(
        _vq_block,
        grid=(nblocks,),
        in_specs=[
            pl.BlockSpec((_ROWS_PER_BLOCK, dim), lambda i: (i, 0)),
            pl.BlockSpec((dim, k), lambda i: (0, 0)),
        ],
        out_specs=[
            pl.BlockSpec((1, 1, _ROWS_PER_BLOCK), lambda i: (i, 0, 0)),
            pl.BlockSpec((1, 1), lambda i: (0, 0)),
        ],
        out_shape=[
            jax.ShapeDtypeStruct((nblocks, 1, _ROWS_PER_BLOCK), jnp.int32),
            jax.ShapeDtypeStruct((1, 1), jnp.float32),
        ],
    )(x, embed)
    return ind3.reshape(n), acc


def _sc_gather(table, idx, n, dim):
    bpw = n // _NW
    mesh = plsc.VectorSubcoreMesh(core_axis_name="c", subcore_axis_name="s")

    @functools.partial(
        pl.kernel,
        mesh=mesh,
        out_type=jax.ShapeDtypeStruct((n, dim), jnp.float32),
        scratch_types=[
            pltpu.VMEM((bpw,), jnp.int32),
            pltpu.VMEM((bpw, dim), jnp.float32),
            pltpu.SemaphoreType.DMA,
        ],
        compiler_params=pltpu.CompilerParams(use_tc_tiling_on_sc=False),
    )
    def gather_k(table_hbm, idx_hbm, out_hbm, idx_v, rows_v, sem):
        wid = lax.axis_index("s") * _NC + lax.axis_index("c")
        base = wid * bpw
        pltpu.sync_copy(idx_hbm.at[pl.ds(base, bpw)], idx_v)
        copies = []
        for j in range(bpw // _IDX_CHUNK):
            copies.append(pltpu.async_copy(
                table_hbm.at[idx_v.at[pl.ds(j * _IDX_CHUNK, _IDX_CHUNK)]],
                rows_v.at[pl.ds(j * _IDX_CHUNK, _IDX_CHUNK)],
                sem,
            ))
        for c in copies:
            c.wait()
        pltpu.sync_copy(rows_v, out_hbm.at[pl.ds(base, bpw)])

    return gather_k(table, idx)


@functools.partial(jax.jit, static_argnames=())
def kernel(input, embed):
    n = input.shape[0] * input.shape[1]
    dim = embed.shape[0]
    x = input.reshape(n, dim)
    ind_flat, acc = _tc_argmin(x, embed)
    q = _sc_gather(embed.T.copy(), ind_flat, n, dim)
    quantize = q.reshape(input.shape)
    embed_ind = ind_flat.reshape(input.shape[:-1])
    diff = (acc[0, 0] / (n * dim)).astype(jnp.float32)
    return (quantize, diff, embed_ind)


# speedup vs baseline: 1.1560x; 1.0163x over previous
import functools

import jax
import jax.numpy as jnp
from jax import lax
from jax.experimental import pallas as pl
from jax.experimental.pallas import tpu as pltpu
from jax.experimental.pallas import tpu_sc as plsc

_ROWS_PER_BLOCK = 4096

_SC_INFO = plsc.get_sparse_core_info()
_NC = _SC_INFO.num_cores
_NS = _SC_INFO.num_subcores
_NW = _NC * _NS
_IDX_CHUNK = 128


def _vq_block(x_ref, e_ref, ind_ref, acc_ref):
    i = pl.program_id(0)
    x = x_ref[...]
    e = e_ref[...]
    s = jax.lax.dot_general(
        x * -2.0, e, (((1,), (0,)), ((), ())),
        preferred_element_type=jnp.float32,
    )
    x2 = jnp.sum(x * x, axis=1, keepdims=True)
    e2 = jnp.sum(e * e, axis=0, keepdims=True)
    dist = x2 + s + e2
    ind = jnp.argmin(dist, axis=1).astype(jnp.int32)
    mind = jnp.min(dist, axis=1)
    ind_ref[0, 0, :] = ind

    @pl.when(i == 0)
    def _init():
        acc_ref[...] = jnp.zeros((1, 1), jnp.float32)

    acc_ref[...] += jnp.sum(mind).reshape(1, 1)


def _tc_argmin(x, embed):
    n, dim = x.shape
    k = embed.shape[1]
    nblocks = n // _ROWS_PER_BLOCK
    ind3, acc = pl.pallas_call(
        _vq_block,
        grid=(nblocks,),
        in_specs=[
            pl.BlockSpec((_ROWS_PER_BLOCK, dim), lambda i: (i, 0)),
            pl.BlockSpec((dim, k), lambda i: (0, 0)),
        ],
        out_specs=[
            pl.BlockSpec((1, 1, _ROWS_PER_BLOCK), lambda i: (i, 0, 0)),
            pl.BlockSpec((1, 1), lambda i: (0, 0)),
        ],
        out_shape=[
            jax.ShapeDtypeStruct((nblocks, 1, _ROWS_PER_BLOCK), jnp.int32),
            jax.ShapeDtypeStruct((1, 1), jnp.float32),
        ],
    )(x, embed)
    return ind3.reshape(n), acc


def _sc_gather(table, idx, n, dim):
    bpw = n // _NS
    mesh = plsc.VectorSubcoreMesh(
        core_axis_name="c", subcore_axis_name="s", num_cores=1)

    @functools.partial(
        pl.kernel,
        mesh=mesh,
        out_type=jax.ShapeDtypeStruct((n, dim), jnp.float32),
        scratch_types=[
            pltpu.VMEM((bpw,), jnp.int32),
            pltpu.VMEM((bpw, dim), jnp.float32),
            pltpu.SemaphoreType.DMA,
        ],
        compiler_params=pltpu.CompilerParams(use_tc_tiling_on_sc=False),
    )
    def gather_k(table_hbm, idx_hbm, out_hbm, idx_v, rows_v, sem):
        wid = lax.axis_index("s")
        base = wid * bpw
        pltpu.sync_copy(idx_hbm.at[pl.ds(base, bpw)], idx_v)
        copies = []
        for j in range(bpw // _IDX_CHUNK):
            copies.append(pltpu.async_copy(
                table_hbm.at[idx_v.at[pl.ds(j * _IDX_CHUNK, _IDX_CHUNK)]],
                rows_v.at[pl.ds(j * _IDX_CHUNK, _IDX_CHUNK)],
                sem,
            ))
        for c in copies:
            c.wait()
        pltpu.sync_copy(rows_v, out_hbm.at[pl.ds(base, bpw)])

    return gather_k(table, idx)


@functools.partial(jax.jit, static_argnames=())
def kernel(input, embed):
    n = input.shape[0] * input.shape[1]
    dim = embed.shape[0]
    x = input.reshape(n, dim)
    ind_flat, acc = _tc_argmin(x, embed)
    q = _sc_gather(embed.T.copy(), ind_flat, n, dim)
    quantize = q.reshape(input.shape)
    embed_ind = ind_flat.reshape(input.shape[:-1])
    diff = (acc[0, 0] / (n * dim)).astype(jnp.float32)
    return (quantize, diff, embed_ind)


# speedup vs baseline: 2.2331x; 1.9318x over previous
import functools

import jax
import jax.numpy as jnp
from jax.experimental import pallas as pl

_ROWS_PER_BLOCK = 4096


def _vq_block(x_ref, e_ref, q_ref, ind_ref, acc_ref):
    i = pl.program_id(0)
    x = x_ref[...]
    e = e_ref[...]
    s = jax.lax.dot_general(
        x * -2.0, e, (((1,), (0,)), ((), ())),
        preferred_element_type=jnp.float32,
    )
    x2 = jnp.sum(x * x, axis=1, keepdims=True)
    e2 = jnp.sum(e * e, axis=0, keepdims=True)
    dist = x2 + s + e2
    ind = jnp.argmin(dist, axis=1).astype(jnp.int32)
    iota = jax.lax.broadcasted_iota(jnp.int32, dist.shape, 1)
    onehot = (iota == ind[:, None]).astype(jnp.float32)
    q = jax.lax.dot_general(
        onehot, e, (((1,), (1,)), ((), ())),
        preferred_element_type=jnp.float32,
    )
    q_ref[...] = q
    ind_ref[0, 0, :] = ind
    r = q - x

    @pl.when(i == 0)
    def _init():
        acc_ref[...] = jnp.zeros((1, 1), jnp.float32)

    acc_ref[...] += jnp.sum(r * r).reshape(1, 1)


@functools.partial(jax.jit, static_argnames=())
def kernel(input, embed):
    n = input.shape[0] * input.shape[1]
    dim = embed.shape[0]
    k = embed.shape[1]
    x = input.reshape(n, dim)
    nblocks = n // _ROWS_PER_BLOCK
    q, ind3, acc = pl.pallas_call(
        _vq_block,
        grid=(nblocks,),
        in_specs=[
            pl.BlockSpec((_ROWS_PER_BLOCK, dim), lambda i: (i, 0)),
            pl.BlockSpec((dim, k), lambda i: (0, 0)),
        ],
        out_specs=[
            pl.BlockSpec((_ROWS_PER_BLOCK, dim), lambda i: (i, 0)),
            pl.BlockSpec((1, 1, _ROWS_PER_BLOCK), lambda i: (i, 0, 0)),
            pl.BlockSpec((1, 1), lambda i: (0, 0)),
        ],
        out_shape=[
            jax.ShapeDtypeStruct((n, dim), jnp.float32),
            jax.ShapeDtypeStruct((nblocks, 1, _ROWS_PER_BLOCK), jnp.int32),
            jax.ShapeDtypeStruct((1, 1), jnp.float32),
        ],
    )(x, embed)
    quantize = q.reshape(input.shape)
    embed_ind = ind3.reshape(input.shape[:-1])
    diff = (acc[0, 0] / (n * dim)).astype(jnp.float32)
    return (quantize, diff, embed_ind)
